# baseline (speedup 1.0000x reference)
import functools

import jax
import jax.numpy as jnp
from jax import lax
from jax.experimental import pallas as pl
from jax.experimental.pallas import tpu as pltpu
from jax.experimental.pallas import tpu_sc as plsc

NUM_CODES_C = 8192
CODE_D = 256
BLOCK_N = 128


def _vq_body(x_ref, e_ref, dist_ref, ind_ref, oh_ref, e2_ref):
    x = x_ref[...]
    e = e_ref[...]

    @pl.when(pl.program_id(0) == 0)
    def _():
        e2_ref[...] = jnp.sum(e * e, axis=1)[None, :]

    cross = jax.lax.dot_general(
        x, e, (((1,), (1,)), ((), ())),
        preferred_element_type=jnp.float32)
    f2 = jnp.sum(x * x, axis=1, keepdims=True)
    dist = -(f2 - 2.0 * cross + e2_ref[...])
    dist_ref[...] = dist
    m = jnp.max(dist, axis=1, keepdims=True)
    iota_row = jax.lax.broadcasted_iota(jnp.int32, (1, dist.shape[1]), 1)
    cand = jnp.where(dist == m, iota_row, NUM_CODES_C)
    ind = jnp.min(cand, axis=1)
    ind_ref[0, 0, :] = ind
    oh_ref[...] = (iota_row == ind[:, None]).astype(jnp.float32)


@jax.jit
def _vq_tc(x2d, e2d):
    n, d = x2d.shape
    c = e2d.shape[0]
    nb = n // BLOCK_N
    return pl.pallas_call(
        _vq_body,
        grid=(nb,),
        in_specs=[
            pl.BlockSpec((BLOCK_N, d), lambda i: (i, 0)),
            pl.BlockSpec((c, d), lambda i: (0, 0)),
        ],
        out_specs=[
            pl.BlockSpec((BLOCK_N, c), lambda i: (i, 0)),
            pl.BlockSpec((1, 1, BLOCK_N), lambda i: (i, 0, 0)),
            pl.BlockSpec((BLOCK_N, c), lambda i: (i, 0)),
        ],
        out_shape=[
            jax.ShapeDtypeStruct((n, c), jnp.float32),
            jax.ShapeDtypeStruct((nb, 1, BLOCK_N), jnp.int32),
            jax.ShapeDtypeStruct((n, c), jnp.float32),
        ],
        scratch_shapes=[pltpu.VMEM((1, c), jnp.float32)],
        compiler_params=pltpu.CompilerParams(
            dimension_semantics=("arbitrary",),
            vmem_limit_bytes=112 * 1024 * 1024),
    )(x2d, e2d)


def _sc_gather(e2d, idx3):
    info = plsc.get_sparse_core_info()
    nc, ns = info.num_cores, info.num_subcores
    nw = nc * ns
    chunks, ilen = idx3.shape[1], idx3.shape[2]
    b_per_w = chunks * ilen
    d = e2d.shape[1]
    mesh = plsc.VectorSubcoreMesh(core_axis_name="c", subcore_axis_name="s")

    @functools.partial(
        pl.kernel, mesh=mesh,
        out_type=jax.ShapeDtypeStruct((nw * b_per_w, d), jnp.float32),
        scratch_types=[
            pltpu.VMEM((chunks, ilen), jnp.int32),
            pltpu.VMEM((b_per_w, d), jnp.float32),
            pltpu.SemaphoreType.DMA,
        ],
    )
    def gather_k(table_hbm, idx_hbm, out_hbm, idx_v, rows_v, sem):
        wid = lax.axis_index("s") * nc + lax.axis_index("c")
        pltpu.sync_copy(idx_hbm.at[wid], idx_v)
        copies = [
            pltpu.async_copy(
                table_hbm.at[idx_v.at[j]],
                rows_v.at[pl.ds(j * ilen, ilen)], sem)
            for j in range(chunks)
        ]
        for cpy in copies:
            cpy.wait()
        pltpu.sync_copy(rows_v, out_hbm.at[pl.ds(wid * b_per_w, b_per_w)])

    return gather_k(e2d, idx3)


@jax.jit
def _vq(x2d, e2d):
    dist2d, ind3, onehot = _vq_tc(x2d, e2d)
    ind_flat = ind3.reshape(-1)
    quant2d = _sc_gather(e2d, ind_flat.reshape(32, -1, 128))
    return dist2d, ind_flat, onehot, quant2d


def kernel(x, embed):
    x = x.astype(jnp.float32)
    orig_shape = x.shape
    d = x.shape[-1]
    c = embed.shape[1]
    x2d = x.reshape(-1, d)
    e2d = embed.reshape(c, d)
    dist2d, ind_flat, onehot, quant2d = _vq(x2d, e2d)
    quantize = quant2d.reshape(orig_shape)
    embed_ind = ind_flat.reshape(orig_shape[:-1])
    embed_onehot = onehot[None]
    dist_out = dist2d.reshape((1,) + orig_shape[:-1] + (c,))
    return (quantize, embed_ind, embed_onehot, dist_out)


# speedup vs baseline: 1.2572x; 1.2572x over previous
import functools

import jax
import jax.numpy as jnp
from jax import lax
from jax.experimental import pallas as pl
from jax.experimental.pallas import tpu as pltpu
from jax.experimental.pallas import tpu_sc as plsc

NUM_CODES_C = 8192
CODE_D = 256
BLOCK_N = 256


def _vq_body(x_ref, e_ref, dist_ref, ind_ref, oh_ref, e2_ref):
    x = x_ref[...]
    e = e_ref[...]

    @pl.when(pl.program_id(0) == 0)
    def _():
        e2_ref[...] = jnp.sum(e * e, axis=1)[None, :]

    cross = jax.lax.dot_general(
        x, e, (((1,), (1,)), ((), ())),
        preferred_element_type=jnp.float32)
    f2 = jnp.sum(x * x, axis=1, keepdims=True)
    dist = -(f2 - 2.0 * cross + e2_ref[...])
    dist_ref[...] = dist
    m = jnp.max(dist, axis=1, keepdims=True)
    iota_row = jax.lax.broadcasted_iota(jnp.int32, (1, dist.shape[1]), 1)
    cand = jnp.where(dist == m, iota_row, NUM_CODES_C)
    ind = jnp.min(cand, axis=1)
    ind_ref[0, 0, :] = ind
    oh_ref[...] = (iota_row == ind[:, None]).astype(jnp.float32)


@jax.jit
def _vq_tc(x2d, e2d):
    n, d = x2d.shape
    c = e2d.shape[0]
    nb = n // BLOCK_N
    return pl.pallas_call(
        _vq_body,
        grid=(nb,),
        in_specs=[
            pl.BlockSpec((BLOCK_N, d), lambda i: (i, 0)),
            pl.BlockSpec((c, d), lambda i: (0, 0)),
        ],
        out_specs=[
            pl.BlockSpec((BLOCK_N, c), lambda i: (i, 0)),
            pl.BlockSpec((1, 1, BLOCK_N), lambda i: (i, 0, 0)),
            pl.BlockSpec((BLOCK_N, c), lambda i: (i, 0)),
        ],
        out_shape=[
            jax.ShapeDtypeStruct((n, c), jnp.float32),
            jax.ShapeDtypeStruct((nb, 1, BLOCK_N), jnp.int32),
            jax.ShapeDtypeStruct((n, c), jnp.float32),
        ],
        scratch_shapes=[pltpu.VMEM((1, c), jnp.float32)],
        compiler_params=pltpu.CompilerParams(
            dimension_semantics=("arbitrary",),
            vmem_limit_bytes=112 * 1024 * 1024),
    )(x2d, e2d)


def _sc_gather(e2d, idx3):
    info = plsc.get_sparse_core_info()
    nc, ns = info.num_cores, info.num_subcores
    nw = nc * ns
    b_per_w = idx3.shape[2]
    ilen = 128
    chunks = b_per_w // ilen
    d = e2d.shape[1]
    mesh = plsc.VectorSubcoreMesh(core_axis_name="c", subcore_axis_name="s")

    @functools.partial(
        pl.kernel, mesh=mesh,
        out_type=jax.ShapeDtypeStruct((nw * b_per_w, d), jnp.float32),
        scratch_types=[
            pltpu.VMEM((b_per_w,), jnp.int32),
            pltpu.VMEM((b_per_w, d), jnp.float32),
            pltpu.SemaphoreType.DMA,
        ],
    )
    def gather_k(table_hbm, idx_hbm, out_hbm, idx_v, rows_v, sem):
        wid = lax.axis_index("s") * nc + lax.axis_index("c")
        pltpu.sync_copy(idx_hbm.at[wid, 0], idx_v)
        copies = [
            pltpu.async_copy(
                table_hbm.at[idx_v.at[pl.ds(j * ilen, ilen)]],
                rows_v.at[pl.ds(j * ilen, ilen)], sem)
            for j in range(chunks)
        ]
        for cpy in copies:
            cpy.wait()
        pltpu.sync_copy(rows_v, out_hbm.at[pl.ds(wid * b_per_w, b_per_w)])

    return gather_k(e2d, idx3)


@functools.partial(jax.jit, static_argnames=("orig_shape",))
def _vq(x2d, e2d, orig_shape):
    c = e2d.shape[0]
    dist2d, ind3, onehot = _vq_tc(x2d, e2d)
    quant2d = _sc_gather(e2d, ind3)
    quantize = quant2d.reshape(orig_shape)
    embed_ind = ind3.reshape(orig_shape[:-1])
    embed_onehot = onehot[None]
    dist_out = dist2d.reshape((1,) + orig_shape[:-1] + (c,))
    return quantize, embed_ind, embed_onehot, dist_out


def kernel(x, embed):
    x = x.astype(jnp.float32)
    d = x.shape[-1]
    c = embed.shape[1]
    x2d = x.reshape(-1, d)
    e2d = embed.reshape(c, d)
    return _vq(x2d, e2d, x.shape)
